# baseline (speedup 1.0000x reference)
import functools

import jax
import jax.numpy as jnp
from jax import lax
from jax.experimental import pallas as pl
from jax.experimental.pallas import tpu as pltpu
from jax.experimental.pallas import tpu_sc as plsc

D = 64
NC = 2
NS = 16
NW = NC * NS
L = 16
CHUNK = 128
HIST = 200
BATCH = 4096
NTJ = BATCH // 128
NQ = HIST * NTJ
NPW = NQ // NW


def _body(tbl_hbm, xt_hbm, out_hbm, idx_raw, idx_half, idx_par, rows_v, t_v,
          sem_g0, sem_g1, sem_s0, sem_s1):
    wid = lax.axis_index("s") * NC + lax.axis_index("c")
    sem_g = (sem_g0, sem_g1)
    sem_s = (sem_s0, sem_s1)
    iota = lax.broadcasted_iota(jnp.int32, (L,), 0)

    def load_prep(q, b):
        h = q // NTJ
        tj = q % NTJ
        pltpu.sync_copy(xt_hbm.at[h // 8, tj, h % 8, :], idx_raw.at[b])
        for m in range(CHUNK // L):
            v = idx_raw[b, pl.ds(m * L, L)]
            idx_half[b, pl.ds(m * L, L)] = lax.shift_right_logical(v, 1)
            idx_par[b, pl.ds(m * L, L)] = lax.shift_left(
                lax.bitwise_and(v, 1), 6)

    def gather_cp(b):
        return pltpu.make_async_copy(
            tbl_hbm.at[idx_half.at[b]], rows_v.at[b], sem_g[b])

    def store_cp(q, b):
        h = q // NTJ
        tj = q % NTJ
        return pltpu.make_async_copy(
            t_v.at[b], out_hbm.at[h, :, tj, :, :], sem_s[b])

    def transpose(b):
        def tg(g, carry):
            g16 = g * L
            b16 = iota + g16
            par16 = idx_par[b, pl.ds(g16, L)]
            for d in range(D):
                v = plsc.load_gather(rows_v.at[b], [b16, par16 + d])
                t_v[b, d // 8, d % 8, pl.ds(g16, L)] = v
            return carry

        lax.fori_loop(0, CHUNK // L, tg, 0)

    def chunk_q(i):
        return i * NW + wid

    for b in (0, 1):
        load_prep(chunk_q(b), b)
        gather_cp(b).start()
    for b in (0, 1):
        gather_cp(b).wait()
        transpose(b)
        store_cp(chunk_q(b), b).start()
        load_prep(chunk_q(b + 2), b)
        gather_cp(b).start()

    def step2(g, carry):
        for b in (0, 1):
            i = 2 * g + b
            gather_cp(b).wait()
            store_cp(chunk_q(i - 2), b).wait()
            transpose(b)
            store_cp(chunk_q(i), b).start()
            load_prep(chunk_q(i + 2), b)
            gather_cp(b).start()
        return carry

    lax.fori_loop(1, NPW // 2 - 1, step2, 0)

    for b in (0, 1):
        i = NPW - 2 + b
        gather_cp(b).wait()
        store_cp(chunk_q(i - 2), b).wait()
        transpose(b)
        store_cp(chunk_q(i), b).start()
    for b in (0, 1):
        store_cp(chunk_q(NPW - 2 + b), b).wait()


@jax.jit
def _gather(tbl2, xt4):
    mesh = plsc.VectorSubcoreMesh(core_axis_name="c", subcore_axis_name="s")
    return pl.kernel(
        _body,
        out_type=jax.ShapeDtypeStruct((HIST, D // 8, NTJ, 8, 128), jnp.float32),
        mesh=mesh,
        scratch_types=[
            pltpu.VMEM((2, CHUNK), jnp.int32),
            pltpu.VMEM((2, CHUNK), jnp.int32),
            pltpu.VMEM((2, CHUNK), jnp.int32),
            pltpu.VMEM((2, CHUNK, 128), jnp.float32),
            pltpu.VMEM((2, D // 8, 8, 128), jnp.float32),
            pltpu.SemaphoreType.DMA,
            pltpu.SemaphoreType.DMA,
            pltpu.SemaphoreType.DMA,
            pltpu.SemaphoreType.DMA,
        ],
        compiler_params=pltpu.CompilerParams(
            use_tc_tiling_on_sc=False, needs_layout_passes=False),
    )(tbl2, xt4)


def kernel(x, table):
    b, h = x.shape
    xt4 = x.T.reshape(h // 8, 8, b // 128, 128).transpose(0, 2, 1, 3)
    tbl2 = table.reshape(table.shape[0] // 2, 2 * D)
    out5 = _gather(tbl2, xt4)
    return out5.transpose(2, 4, 0, 1, 3).reshape(b, h, D)


# speedup vs baseline: 1.7035x; 1.7035x over previous
import functools

import jax
import jax.numpy as jnp
from jax import lax
from jax.experimental import pallas as pl
from jax.experimental.pallas import tpu as pltpu
from jax.experimental.pallas import tpu_sc as plsc

D = 64
NC = 2
NS = 16
NW = NC * NS
L = 16
CHUNK = 128
HIST = 200
BATCH = 4096
NTJ = BATCH // 128
NQ = HIST * NTJ
NPW = NQ // NW


def _body(tbl_hbm, xt_hbm, out_hbm, idx_raw, idx_half, idx_par, rows_v, t_v,
          sem_g0, sem_g1, sem_s0, sem_s1):
    wid = lax.axis_index("s") * NC + lax.axis_index("c")
    sem_g = (sem_g0, sem_g1)
    sem_s = (sem_s0, sem_s1)
    iota = lax.broadcasted_iota(jnp.int32, (L,), 0)

    def load_prep(q, b):
        h = q // NTJ
        tj = q % NTJ
        pltpu.sync_copy(xt_hbm.at[h // 8, tj, h % 8, :], idx_raw.at[b])
        for m in range(CHUNK // L):
            v = idx_raw[b, pl.ds(m * L, L)]
            idx_half[b, pl.ds(m * L, L)] = lax.shift_right_logical(v, 1)
            idx_par[b, pl.ds(m * L, L)] = lax.shift_left(
                lax.bitwise_and(v, 1), 6)

    def gather_cp(b):
        return pltpu.make_async_copy(
            tbl_hbm.at[idx_half.at[b]], rows_v.at[b], sem_g[b])

    def store_cp(q, b):
        h = q // NTJ
        tj = q % NTJ
        return pltpu.make_async_copy(
            t_v.at[b], out_hbm.at[h, :, tj, :, :], sem_s[b])

    def transpose(b):
        pars = tuple(idx_par[b, pl.ds(g * L, L)] for g in range(CHUNK // L))

        def td(dd, pars):
            d16 = lax.bitwise_and(dd + iota, D - 1)
            ti16 = lax.shift_right_logical(d16, 3)
            r16 = lax.bitwise_and(d16, 7)
            for g in range(CHUNK // L):
                v = plsc.load_gather(rows_v.at[b], [iota + g * L, pars[g] + d16])
                plsc.store_scatter(t_v.at[b], [ti16, r16, iota + g * L], v)
            return pars

        lax.fori_loop(0, D, td, pars)

    def chunk_q(i):
        return i * NW + wid

    for b in (0, 1):
        load_prep(chunk_q(b), b)
        gather_cp(b).start()
    for b in (0, 1):
        gather_cp(b).wait()
        transpose(b)
        store_cp(chunk_q(b), b).start()
        load_prep(chunk_q(b + 2), b)
        gather_cp(b).start()

    def step2(g, carry):
        for b in (0, 1):
            i = 2 * g + b
            gather_cp(b).wait()
            store_cp(chunk_q(i - 2), b).wait()
            transpose(b)
            store_cp(chunk_q(i), b).start()
            load_prep(chunk_q(i + 2), b)
            gather_cp(b).start()
        return carry

    lax.fori_loop(1, NPW // 2 - 1, step2, 0)

    for b in (0, 1):
        i = NPW - 2 + b
        gather_cp(b).wait()
        store_cp(chunk_q(i - 2), b).wait()
        transpose(b)
        store_cp(chunk_q(i), b).start()
    for b in (0, 1):
        store_cp(chunk_q(NPW - 2 + b), b).wait()


@jax.jit
def _gather(tbl2, xt4):
    mesh = plsc.VectorSubcoreMesh(core_axis_name="c", subcore_axis_name="s")
    return pl.kernel(
        _body,
        out_type=jax.ShapeDtypeStruct((HIST, D // 8, NTJ, 8, 128), jnp.float32),
        mesh=mesh,
        scratch_types=[
            pltpu.VMEM((2, CHUNK), jnp.int32),
            pltpu.VMEM((2, CHUNK), jnp.int32),
            pltpu.VMEM((2, CHUNK), jnp.int32),
            pltpu.VMEM((2, CHUNK, 128), jnp.float32),
            pltpu.VMEM((2, D // 8, 8, 128), jnp.float32),
            pltpu.SemaphoreType.DMA,
            pltpu.SemaphoreType.DMA,
            pltpu.SemaphoreType.DMA,
            pltpu.SemaphoreType.DMA,
        ],
        compiler_params=pltpu.CompilerParams(
            use_tc_tiling_on_sc=False, needs_layout_passes=False),
    )(tbl2, xt4)


def kernel(x, table):
    b, h = x.shape
    xt4 = x.T.reshape(h // 8, 8, b // 128, 128).transpose(0, 2, 1, 3)
    tbl2 = table.reshape(table.shape[0] // 2, 2 * D)
    out5 = _gather(tbl2, xt4)
    return out5.transpose(2, 4, 0, 1, 3).reshape(b, h, D)
